# baseline (speedup 1.0000x reference)
import jax
import jax.numpy as jnp
import numpy as np
from jax.experimental import pallas as pl
from jax.experimental.pallas import tpu as pltpu

_B = 64
_M = 128
_F = 48
_C = 7
_H = 20
_N = _B * _M

_GP = 2

_SQRT2 = 1.4142135623730951

_G_BLOCK = np.kron(np.eye(_C, dtype=np.float32),
                   np.ones((_C, _C), dtype=np.float32))


def _gelu(x):
    return 0.5 * x * (1.0 + jax.lax.erf(x / _SQRT2))


def _elu(x):
    return jnp.maximum(x, 0.0) + (jnp.exp(jnp.minimum(x, 0.0)) - 1.0)


def _b16(x):
    return x.astype(jnp.bfloat16)


def _dinv(Ab, ones_col):
    deg = jax.lax.dot_general(Ab, ones_col, (((0,), (0,)), ((), ())),
                              preferred_element_type=jnp.float32)
    safe = jnp.where(deg > 0, deg, 1.0)
    return jnp.where(deg > 0, jax.lax.rsqrt(safe), 0.0)


def _dot(a, b):
    return jnp.dot(_b16(a), b, preferred_element_type=jnp.float32)


def _aggT(An_b, d, h):
    return d * jax.lax.dot_general(
        An_b, _b16(h), (((0,), (0,)), ((), ())),
        preferred_element_type=jnp.float32)


def _fused_kernel(Aa_ref, Aw_ref, Ae_ref, Al_ref,
                  xT_ref, xTw_ref, xTe_ref, xD_ref, xDu_ref, xL_ref,
                  WV1_ref, b1a_ref, b1s_ref, M2a_ref, M2s_ref,
                  b2a_ref, b2s_ref,
                  L1_ref, bl1_ref, L2_ref, bl2_ref,
                  G_ref, P_ref, bf_ref,
                  out_ref):
    G = G_ref[...]
    ones_col = jnp.full((_M, 1), 1.0, dtype=jnp.bfloat16)
    for i in range(_GP):
        Aa = Aa_ref[i]
        Aw = Aw_ref[i]
        Ae = Ae_ref[i]
        Al = Al_ref[i]
        da = _dinv(_b16(Aa), ones_col)
        dw = _dinv(_b16(Aw), ones_col)
        de = _dinv(_b16(Ae), ones_col)
        dl = _dinv(_b16(Al), ones_col)
        Aa = _b16(da * Aa)
        Aw = _b16(dw * Aw)
        Ae = _b16(de * Ae)
        Al = _b16(dl * Al)

        hT = _dot(xT_ref[i], WV1_ref[0])
        hD = _dot(xD_ref[i], WV1_ref[1])
        hDu = _dot(xDu_ref[i], WV1_ref[2])
        hW = _dot(xTw_ref[i], WV1_ref[3])
        hE = _dot(xTe_ref[i], WV1_ref[4])
        hL = _dot(xL_ref[i], WV1_ref[5])

        hw_a = jnp.concatenate([hT[:, :_H], hD[:, :_H], hDu[:, :_H]], axis=1)
        hv_a = jnp.concatenate([hT[:, _H:], hD[:, _H:], hDu[:, _H:]], axis=1)
        h1a = _elu(_gelu(_aggT(Aa, da, hw_a) + hv_a + b1a_ref[...]))
        h1w = _elu(_gelu(_aggT(Aw, dw, hW[:, :_H]) + hW[:, _H:] + b1s_ref[0]))
        h1e = _elu(_gelu(_aggT(Ae, de, hE[:, :_H]) + hE[:, _H:] + b1s_ref[1]))
        h1l = _elu(_gelu(_aggT(Al, dl, hL[:, :_H]) + hL[:, _H:] + b1s_ref[2]))

        H2a = _dot(h1a, M2a_ref[...])
        H2w = _dot(h1w, M2s_ref[0])
        H2e = _dot(h1e, M2s_ref[1])
        H2l = _dot(h1l, M2s_ref[2])
        s_a = jax.nn.relu(_aggT(Aa, da, H2a[:, :3 * _C])
                          + H2a[:, 3 * _C:] + b2a_ref[...])
        s_w = jax.nn.relu(_aggT(Aw, dw, H2w[:, :_C]) + H2w[:, _C:]
                          + b2s_ref[0])
        s_e = jax.nn.relu(_aggT(Ae, de, H2e[:, :_C]) + H2e[:, _C:]
                          + b2s_ref[1])
        s_l = jax.nn.relu(_aggT(Al, dl, H2l[:, :_C]) + H2l[:, _C:]
                          + b2s_ref[2])

        t = jax.nn.relu(_dot(xL_ref[i], L1_ref[...]) + bl1_ref[...])
        s_lt = _dot(t, L2_ref[...]) + bl2_ref[...]

        S = jnp.concatenate([s_a, s_w, s_e, s_l, s_lt], axis=1)
        E = jnp.exp(S - jnp.max(S, axis=-1, keepdims=True))
        En = E / _dot(E, G)
        out_ref[pl.ds(i * _M, _M), :] = _dot(En, P_ref[...]) + bf_ref[...]


def kernel(A_input, A_week_input, A_weekend_input, Location_location_input,
           Temporal_input, Temporal_week_input, Temporal_weekend_input,
           Distance_input, Duration_input, Location_time_input,
           W1_temporal, V1_temporal, b1_temporal, W2_temporal, V2_temporal, b2_temporal,
           W1_week, V1_week, b1_week, W2_week, V2_week, b2_week,
           W1_weekend, V1_weekend, b1_weekend, W2_weekend, V2_weekend, b2_weekend,
           W1_distance, V1_distance, b1_distance, W2_distance, V2_distance, b2_distance,
           W1_duration, V1_duration, b1_duration, W2_duration, V2_duration, b2_duration,
           W1_loctime, V1_loctime, b1_loctime, W2_loctime, V2_loctime, b2_loctime,
           L1, bl1, L2, bl2, Ld, bd, Lo, bo):
    WV1 = jnp.stack([
        jnp.concatenate([W1_temporal, V1_temporal], axis=1),
        jnp.concatenate([W1_distance, V1_distance], axis=1),
        jnp.concatenate([W1_duration, V1_duration], axis=1),
        jnp.concatenate([W1_week, V1_week], axis=1),
        jnp.concatenate([W1_weekend, V1_weekend], axis=1),
        jnp.concatenate([W1_loctime, V1_loctime], axis=1),
    ])
    b1a = jnp.concatenate([b1_temporal, b1_distance, b1_duration])[None, :]
    b1s = jnp.stack([b1_week, b1_weekend, b1_loctime])[:, None, :]

    z = jnp.zeros((_H, _C), jnp.float32)
    M2a = jnp.concatenate([
        jnp.concatenate([W2_temporal, z, z, V2_temporal, z, z], axis=1),
        jnp.concatenate([z, W2_distance, z, z, V2_distance, z], axis=1),
        jnp.concatenate([z, z, W2_duration, z, z, V2_duration], axis=1),
    ], axis=0)
    M2s = jnp.stack([
        jnp.concatenate([W2_week, V2_week], axis=1),
        jnp.concatenate([W2_weekend, V2_weekend], axis=1),
        jnp.concatenate([W2_loctime, V2_loctime], axis=1),
    ])
    b2a = jnp.concatenate([b2_temporal, b2_distance, b2_duration])[None, :]
    b2s = jnp.stack([b2_week, b2_weekend, b2_loctime])[:, None, :]

    P = jnp.concatenate([Lo, Lo, Lo, Lo, Lo, 2.0 * Ld, 2.0 * Ld], axis=0)
    bf = (bd + bo)[None, :]

    grid = (_B // _GP,)
    badj = pl.BlockSpec((_GP, _M, _M), lambda b: (b, 0, 0))
    bx = pl.BlockSpec((_GP, _M, _F), lambda b: (b, 0, 0))

    def bcast(shape):
        nd = len(shape)
        return pl.BlockSpec(shape, lambda b: (0,) * nd)

    out = pl.pallas_call(
        _fused_kernel,
        grid=grid,
        in_specs=[badj, badj, badj, badj,
                  bx, bx, bx, bx, bx, bx,
                  bcast((6, _F, 2 * _H)), bcast((1, 3 * _H)),
                  bcast((3, 1, _H)),
                  bcast((3 * _H, 6 * _C)), bcast((3, _H, 2 * _C)),
                  bcast((1, 3 * _C)), bcast((3, 1, _C)),
                  bcast((_F, 40)), bcast((1, 40)),
                  bcast((40, _C)), bcast((1, _C)),
                  bcast((_C * _C, _C * _C)),
                  bcast((_C * _C, _C)), bcast((1, _C))],
        out_specs=pl.BlockSpec((_GP * _M, _C), lambda b: (b, 0)),
        out_shape=jax.ShapeDtypeStruct((_N, _C), jnp.float32),
        compiler_params=pltpu.CompilerParams(
            dimension_semantics=("parallel",)),
    )(A_input, A_week_input, A_weekend_input, Location_location_input,
      Temporal_input, Temporal_week_input, Temporal_weekend_input,
      Distance_input, Duration_input, Location_time_input,
      WV1.astype(jnp.bfloat16), b1a, b1s,
      M2a.astype(jnp.bfloat16), M2s.astype(jnp.bfloat16), b2a, b2s,
      L1.astype(jnp.bfloat16), bl1[None, :],
      L2.astype(jnp.bfloat16), bl2[None, :],
      jnp.asarray(_G_BLOCK, dtype=jnp.bfloat16),
      P.astype(jnp.bfloat16), bf)
    return out


# speedup vs baseline: 1.0407x; 1.0407x over previous
import jax
import jax.numpy as jnp
import numpy as np
from jax.experimental import pallas as pl
from jax.experimental.pallas import tpu as pltpu

_B = 64
_M = 128
_F = 48
_C = 7
_H = 20
_N = _B * _M

_GP = 8

_SQRT2 = 1.4142135623730951

_REAL = np.array([1.0] * _C + [0.0], dtype=np.float32)
_G_BLOCK = np.kron(np.eye(_C, dtype=np.float32),
                   np.outer(np.ones(8, dtype=np.float32), _REAL))


def _gelu(x):
    return 0.5 * x * (1.0 + jax.lax.erf(x / _SQRT2))


def _elu(x):
    return jnp.maximum(x, 0.0) + (jnp.exp(jnp.minimum(x, 0.0)) - 1.0)


def _b16(x):
    return x.astype(jnp.bfloat16)


def _dinv(Ab, ones_row):
    deg = jnp.dot(ones_row, Ab, preferred_element_type=jnp.float32)
    safe = jnp.where(deg > 0, deg, 1.0)
    return jnp.where(deg > 0, jax.lax.rsqrt(safe), 0.0)


def _mm(a, b):
    return jnp.dot(a, b, preferred_element_type=jnp.float32)


def _dotT(Wt, xb):
    return jax.lax.dot_general(Wt, xb, (((1,), (1,)), ((), ())),
                               preferred_element_type=jnp.float32)


def _agg(Ab, d, hT):
    return d * _mm(_b16(hT * d), Ab)


def _fused_kernel(Aa_ref, Aw_ref, Ae_ref, Al_ref,
                  xT_ref, xTw_ref, xTe_ref, xD_ref, xDu_ref, xL_ref,
                  M1aT_ref, b1aT_ref, WV1sT_ref, b1sT_ref,
                  M2aT_ref, b2aT_ref, M2sT_ref, b2sT_ref,
                  L1T_ref, bl1T_ref, L2T_ref, bl2T_ref,
                  G8_ref, P8_ref, bfT_ref,
                  out_ref):
    ones_row = jnp.full((1, _M), 1.0, dtype=jnp.bfloat16)
    for i in range(_GP):
        Ab_a = _b16(Aa_ref[i])
        Ab_w = _b16(Aw_ref[i])
        Ab_e = _b16(Ae_ref[i])
        Ab_l = _b16(Al_ref[i])
        da = _dinv(Ab_a, ones_row)
        dw = _dinv(Ab_w, ones_row)
        de = _dinv(Ab_e, ones_row)
        dl = _dinv(Ab_l, ones_row)

        xa = jnp.concatenate(
            [_b16(xT_ref[i]), _b16(xD_ref[i]), _b16(xDu_ref[i])], axis=1)
        H1a = _dotT(M1aT_ref[...], xa)
        h1a = _elu(_gelu(_agg(Ab_a, da, H1a[0:64]) + H1a[64:128]
                         + b1aT_ref[...]))

        xLb = _b16(xL_ref[i])
        Hw = _dotT(WV1sT_ref[0], _b16(xTw_ref[i]))
        He = _dotT(WV1sT_ref[1], _b16(xTe_ref[i]))
        Hl = _dotT(WV1sT_ref[2], xLb)
        h1w = _elu(_gelu(_agg(Ab_w, dw, Hw[0:32]) + Hw[32:64]
                         + b1sT_ref[0]))
        h1e = _elu(_gelu(_agg(Ab_e, de, He[0:32]) + He[32:64]
                         + b1sT_ref[1]))
        h1l = _elu(_gelu(_agg(Ab_l, dl, Hl[0:32]) + Hl[32:64]
                         + b1sT_ref[2]))

        H2a = _mm(M2aT_ref[...], _b16(h1a))
        s_a = jax.nn.relu(_agg(Ab_a, da, H2a[0:24]) + H2a[24:48]
                          + b2aT_ref[...])
        H2w = _mm(M2sT_ref[0], _b16(h1w))
        H2e = _mm(M2sT_ref[1], _b16(h1e))
        H2l = _mm(M2sT_ref[2], _b16(h1l))
        s_w = jax.nn.relu(_agg(Ab_w, dw, H2w[0:8]) + H2w[8:16] + b2sT_ref[0])
        s_e = jax.nn.relu(_agg(Ab_e, de, H2e[0:8]) + H2e[8:16] + b2sT_ref[1])
        s_l = jax.nn.relu(_agg(Ab_l, dl, H2l[0:8]) + H2l[8:16] + b2sT_ref[2])

        tT = jax.nn.relu(_dotT(L1T_ref[...], xLb) + bl1T_ref[...])
        s_lt = _mm(L2T_ref[...], _b16(tT)) + bl2T_ref[...]

        S = jnp.concatenate([s_a, s_w, s_e, s_l, s_lt], axis=0)
        E = jnp.exp(S - jnp.max(S, axis=0, keepdims=True))
        En = E / _mm(G8_ref[...], _b16(E))
        out_ref[:, pl.ds(i * _M, _M)] = (_mm(P8_ref[...], _b16(En))
                                         + bfT_ref[...])


def _pad_set(shape, *placements):
    z = jnp.zeros(shape, jnp.float32)
    for (r, c), w in placements:
        z = jax.lax.dynamic_update_slice(z, w, (r, c))
    return z


def kernel(A_input, A_week_input, A_weekend_input, Location_location_input,
           Temporal_input, Temporal_week_input, Temporal_weekend_input,
           Distance_input, Duration_input, Location_time_input,
           W1_temporal, V1_temporal, b1_temporal, W2_temporal, V2_temporal, b2_temporal,
           W1_week, V1_week, b1_week, W2_week, V2_week, b2_week,
           W1_weekend, V1_weekend, b1_weekend, W2_weekend, V2_weekend, b2_weekend,
           W1_distance, V1_distance, b1_distance, W2_distance, V2_distance, b2_distance,
           W1_duration, V1_duration, b1_duration, W2_duration, V2_duration, b2_duration,
           W1_loctime, V1_loctime, b1_loctime, W2_loctime, V2_loctime, b2_loctime,
           L1, bl1, L2, bl2, Ld, bd, Lo, bo):
    f32 = jnp.float32
    M1aT = _pad_set((_M, 3 * _F),
                    ((0, 0), W1_temporal.T), ((20, 48), W1_distance.T),
                    ((40, 96), W1_duration.T),
                    ((64, 0), V1_temporal.T), ((84, 48), V1_distance.T),
                    ((104, 96), V1_duration.T))
    b1aT = _pad_set((64, 1), ((0, 0), jnp.concatenate(
        [b1_temporal, b1_distance, b1_duration])[:, None]))
    WV1sT = jnp.stack([
        _pad_set((64, _F), ((0, 0), W.T), ((32, 0), V.T))
        for W, V in [(W1_week, V1_week), (W1_weekend, V1_weekend),
                     (W1_loctime, V1_loctime)]])
    b1sT = jnp.stack([_pad_set((32, 1), ((0, 0), b[:, None]))
                      for b in [b1_week, b1_weekend, b1_loctime]])

    M2aT = _pad_set((48, 64),
                    ((0, 0), W2_temporal.T), ((8, 20), W2_distance.T),
                    ((16, 40), W2_duration.T),
                    ((24, 0), V2_temporal.T), ((32, 20), V2_distance.T),
                    ((40, 40), V2_duration.T))
    b2aT = _pad_set((24, 1), ((0, 0), b2_temporal[:, None]),
                    ((8, 0), b2_distance[:, None]),
                    ((16, 0), b2_duration[:, None]))
    M2sT = jnp.stack([
        _pad_set((16, 32), ((0, 0), W.T), ((8, 0), V.T))
        for W, V in [(W2_week, V2_week), (W2_weekend, V2_weekend),
                     (W2_loctime, V2_loctime)]])
    b2sT = jnp.stack([_pad_set((8, 1), ((0, 0), b[:, None]))
                      for b in [b2_week, b2_weekend, b2_loctime]])

    L1T = L1.T
    bl1T = bl1[:, None]
    L2T = _pad_set((8, 40), ((0, 0), L2.T))
    bl2T = _pad_set((8, 1), ((0, 0), bl2[:, None]))

    LoB = _pad_set((8, 8), ((0, 0), Lo.T))
    LdB = _pad_set((8, 8), ((0, 0), 2.0 * Ld.T))
    P8 = jnp.concatenate([LoB, LoB, LoB, LoB, LoB, LdB, LdB], axis=1)
    bfT = _pad_set((8, 1), ((0, 0), (bd + bo)[:, None]))

    grid = (_B // _GP,)
    badj = pl.BlockSpec((_GP, _M, _M), lambda b: (b, 0, 0))
    bx = pl.BlockSpec((_GP, _M, _F), lambda b: (b, 0, 0))

    def bcast(shape):
        nd = len(shape)
        return pl.BlockSpec(shape, lambda b: (0,) * nd)

    padded = pl.pallas_call(
        _fused_kernel,
        grid=grid,
        in_specs=[badj, badj, badj, badj,
                  bx, bx, bx, bx, bx, bx,
                  bcast((_M, 3 * _F)), bcast((64, 1)),
                  bcast((3, 64, _F)), bcast((3, 32, 1)),
                  bcast((48, 64)), bcast((24, 1)),
                  bcast((3, 16, 32)), bcast((3, 8, 1)),
                  bcast((40, _F)), bcast((40, 1)),
                  bcast((8, 40)), bcast((8, 1)),
                  bcast((56, 56)), bcast((8, 56)), bcast((8, 1))],
        out_specs=pl.BlockSpec((8, _GP * _M), lambda b: (0, b)),
        out_shape=jax.ShapeDtypeStruct((8, _N), f32),
        compiler_params=pltpu.CompilerParams(
            dimension_semantics=("parallel",)),
    )(A_input, A_week_input, A_weekend_input, Location_location_input,
      Temporal_input, Temporal_week_input, Temporal_weekend_input,
      Distance_input, Duration_input, Location_time_input,
      _b16(M1aT), b1aT, _b16(WV1sT), b1sT,
      _b16(M2aT), b2aT, _b16(M2sT), b2sT,
      _b16(L1T), bl1T, _b16(L2T), bl2T,
      jnp.asarray(_G_BLOCK, dtype=jnp.bfloat16), _b16(P8), bfT)
    return padded[:_C].T


# speedup vs baseline: 1.5251x; 1.4655x over previous
import jax
import jax.numpy as jnp
import numpy as np
from jax.experimental import pallas as pl
from jax.experimental.pallas import tpu as pltpu

_B = 64
_M = 128
_F = 48
_C = 7
_H = 20
_N = _B * _M

_GP = 8
_GM = _GP * _M

_SQRT2 = 1.4142135623730951

_REAL = np.array([1.0] * _C + [0.0], dtype=np.float32)
_G_BLOCK = np.kron(np.eye(_C, dtype=np.float32),
                   np.outer(np.ones(8, dtype=np.float32), _REAL))
_SEL = np.kron(np.eye(_GP, dtype=np.float32), np.ones((1, _M), np.float32))


def _gelu(x):
    return 0.5 * x * (1.0 + jax.lax.erf(x / _SQRT2))


def _elu(x):
    return jnp.maximum(x, 0.0) + (jnp.exp(jnp.minimum(x, 0.0)) - 1.0)


def _b16(x):
    return x.astype(jnp.bfloat16)


def _mm(a, b):
    return jnp.dot(a, b, preferred_element_type=jnp.float32)


def _dotT(Wt, xb):
    return jax.lax.dot_general(Wt, xb, (((1,), (1,)), ((), ())),
                               preferred_element_type=jnp.float32)


def _fused_kernel(Aa_ref, Aw_ref, Ae_ref, Al_ref,
                  xT_ref, xTw_ref, xTe_ref, xD_ref, xDu_ref, xL_ref,
                  M1aT_ref, b1aT_ref, WV1sT_ref, b1sT_ref,
                  M2aT_ref, b2aT_ref, M2sT_ref, b2sT_ref,
                  L1T_ref, bl1T_ref, L2T_ref, bl2T_ref,
                  G8_ref, P8_ref, bfT_ref, Sel_ref,
                  out_ref):
    sel = Sel_ref[...]

    def prep(Aref):
        Ab = _b16(Aref[...]).reshape(_GM, _M)
        deg = _mm(sel, Ab)
        safe = jnp.where(deg > 0, deg, 1.0)
        d = jnp.where(deg > 0, jax.lax.rsqrt(safe), 0.0)
        return Ab, d

    Aa, da = prep(Aa_ref)
    Aw, dw = prep(Aw_ref)
    Ae, de = prep(Ae_ref)
    Al, dl = prep(Al_ref)

    def agg(Ab, d, hT):
        parts = []
        for i in range(_GP):
            di = d[i:i + 1]
            p = _mm(_b16(hT[:, i * _M:(i + 1) * _M] * di),
                    Ab[i * _M:(i + 1) * _M])
            parts.append(p * di)
        return jnp.concatenate(parts, axis=1)

    def rs(ref):
        return _b16(ref[...]).reshape(_GM, _F)

    xa = jnp.concatenate([rs(xT_ref), rs(xD_ref), rs(xDu_ref)], axis=1)
    xLb = rs(xL_ref)
    H1a = _dotT(M1aT_ref[...], xa)
    h1a = _elu(_gelu(agg(Aa, da, H1a[0:64]) + H1a[64:128] + b1aT_ref[...]))
    Hw = _dotT(WV1sT_ref[0], rs(xTw_ref))
    He = _dotT(WV1sT_ref[1], rs(xTe_ref))
    Hl = _dotT(WV1sT_ref[2], xLb)
    h1w = _elu(_gelu(agg(Aw, dw, Hw[0:32]) + Hw[32:64] + b1sT_ref[0]))
    h1e = _elu(_gelu(agg(Ae, de, He[0:32]) + He[32:64] + b1sT_ref[1]))
    h1l = _elu(_gelu(agg(Al, dl, Hl[0:32]) + Hl[32:64] + b1sT_ref[2]))

    H2a = _mm(M2aT_ref[...], _b16(h1a))
    s_a = jax.nn.relu(agg(Aa, da, H2a[0:24]) + H2a[24:48] + b2aT_ref[...])
    H2w = _mm(M2sT_ref[0], _b16(h1w))
    H2e = _mm(M2sT_ref[1], _b16(h1e))
    H2l = _mm(M2sT_ref[2], _b16(h1l))
    s_w = jax.nn.relu(agg(Aw, dw, H2w[0:8]) + H2w[8:16] + b2sT_ref[0])
    s_e = jax.nn.relu(agg(Ae, de, H2e[0:8]) + H2e[8:16] + b2sT_ref[1])
    s_l = jax.nn.relu(agg(Al, dl, H2l[0:8]) + H2l[8:16] + b2sT_ref[2])

    tT = jax.nn.relu(_dotT(L1T_ref[...], xLb) + bl1T_ref[...])
    s_lt = _mm(L2T_ref[...], _b16(tT)) + bl2T_ref[...]

    S = jnp.concatenate([s_a, s_w, s_e, s_l, s_lt], axis=0)
    E = jnp.exp(S - jnp.max(S, axis=0, keepdims=True))
    En = E / _mm(G8_ref[...], _b16(E))
    out_ref[...] = _mm(P8_ref[...], _b16(En)) + bfT_ref[...]


def _pad_set(shape, *placements):
    z = jnp.zeros(shape, jnp.float32)
    for (r, c), w in placements:
        z = jax.lax.dynamic_update_slice(z, w, (r, c))
    return z


def kernel(A_input, A_week_input, A_weekend_input, Location_location_input,
           Temporal_input, Temporal_week_input, Temporal_weekend_input,
           Distance_input, Duration_input, Location_time_input,
           W1_temporal, V1_temporal, b1_temporal, W2_temporal, V2_temporal, b2_temporal,
           W1_week, V1_week, b1_week, W2_week, V2_week, b2_week,
           W1_weekend, V1_weekend, b1_weekend, W2_weekend, V2_weekend, b2_weekend,
           W1_distance, V1_distance, b1_distance, W2_distance, V2_distance, b2_distance,
           W1_duration, V1_duration, b1_duration, W2_duration, V2_duration, b2_duration,
           W1_loctime, V1_loctime, b1_loctime, W2_loctime, V2_loctime, b2_loctime,
           L1, bl1, L2, bl2, Ld, bd, Lo, bo):
    f32 = jnp.float32
    M1aT = _pad_set((_M, 3 * _F),
                    ((0, 0), W1_temporal.T), ((20, 48), W1_distance.T),
                    ((40, 96), W1_duration.T),
                    ((64, 0), V1_temporal.T), ((84, 48), V1_distance.T),
                    ((104, 96), V1_duration.T))
    b1aT = _pad_set((64, 1), ((0, 0), jnp.concatenate(
        [b1_temporal, b1_distance, b1_duration])[:, None]))
    WV1sT = jnp.stack([
        _pad_set((64, _F), ((0, 0), W.T), ((32, 0), V.T))
        for W, V in [(W1_week, V1_week), (W1_weekend, V1_weekend),
                     (W1_loctime, V1_loctime)]])
    b1sT = jnp.stack([_pad_set((32, 1), ((0, 0), b[:, None]))
                      for b in [b1_week, b1_weekend, b1_loctime]])

    M2aT = _pad_set((48, 64),
                    ((0, 0), W2_temporal.T), ((8, 20), W2_distance.T),
                    ((16, 40), W2_duration.T),
                    ((24, 0), V2_temporal.T), ((32, 20), V2_distance.T),
                    ((40, 40), V2_duration.T))
    b2aT = _pad_set((24, 1), ((0, 0), b2_temporal[:, None]),
                    ((8, 0), b2_distance[:, None]),
                    ((16, 0), b2_duration[:, None]))
    M2sT = jnp.stack([
        _pad_set((16, 32), ((0, 0), W.T), ((8, 0), V.T))
        for W, V in [(W2_week, V2_week), (W2_weekend, V2_weekend),
                     (W2_loctime, V2_loctime)]])
    b2sT = jnp.stack([_pad_set((8, 1), ((0, 0), b[:, None]))
                      for b in [b2_week, b2_weekend, b2_loctime]])

    L1T = L1.T
    bl1T = bl1[:, None]
    L2T = _pad_set((8, 40), ((0, 0), L2.T))
    bl2T = _pad_set((8, 1), ((0, 0), bl2[:, None]))

    LoB = _pad_set((8, 8), ((0, 0), Lo.T))
    LdB = _pad_set((8, 8), ((0, 0), 2.0 * Ld.T))
    P8 = jnp.concatenate([LoB, LoB, LoB, LoB, LoB, LdB, LdB], axis=1)
    bfT = _pad_set((8, 1), ((0, 0), (bd + bo)[:, None]))

    grid = (_B // _GP,)
    badj = pl.BlockSpec((_GP, _M, _M), lambda b: (b, 0, 0))
    bx = pl.BlockSpec((_GP, _M, _F), lambda b: (b, 0, 0))

    def bcast(shape):
        nd = len(shape)
        return pl.BlockSpec(shape, lambda b: (0,) * nd)

    padded = pl.pallas_call(
        _fused_kernel,
        grid=grid,
        in_specs=[badj, badj, badj, badj,
                  bx, bx, bx, bx, bx, bx,
                  bcast((_M, 3 * _F)), bcast((64, 1)),
                  bcast((3, 64, _F)), bcast((3, 32, 1)),
                  bcast((48, 64)), bcast((24, 1)),
                  bcast((3, 16, 32)), bcast((3, 8, 1)),
                  bcast((40, _F)), bcast((40, 1)),
                  bcast((8, 40)), bcast((8, 1)),
                  bcast((56, 56)), bcast((8, 56)), bcast((8, 1)),
                  bcast((_GP, _GM))],
        out_specs=pl.BlockSpec((8, _GM), lambda b: (0, b)),
        out_shape=jax.ShapeDtypeStruct((8, _N), f32),
        compiler_params=pltpu.CompilerParams(
            dimension_semantics=("parallel",)),
    )(A_input, A_week_input, A_weekend_input, Location_location_input,
      Temporal_input, Temporal_week_input, Temporal_weekend_input,
      Distance_input, Duration_input, Location_time_input,
      _b16(M1aT), b1aT, _b16(WV1sT), b1sT,
      _b16(M2aT), b2aT, _b16(M2sT), b2sT,
      _b16(L1T), bl1T, _b16(L2T), bl2T,
      jnp.asarray(_G_BLOCK, dtype=jnp.bfloat16), _b16(P8), bfT,
      jnp.asarray(_SEL, dtype=jnp.bfloat16))
    return padded[:_C].T


# speedup vs baseline: 1.5577x; 1.0213x over previous
import jax
import jax.numpy as jnp
import numpy as np
from jax.experimental import pallas as pl
from jax.experimental.pallas import tpu as pltpu

_B = 64
_M = 128
_F = 48
_C = 7
_H = 20
_N = _B * _M

_GP = 16
_GM = _GP * _M

_SQRT2 = 1.4142135623730951

_REAL = np.array([1.0] * _C + [0.0], dtype=np.float32)
_G_BLOCK = np.kron(np.eye(_C, dtype=np.float32),
                   np.outer(np.ones(8, dtype=np.float32), _REAL))
_SEL = np.kron(np.eye(_GP, dtype=np.float32), np.ones((1, _M), np.float32))


def _gelu(x):
    return 0.5 * x * (1.0 + jax.lax.erf(x / _SQRT2))


def _elu(x):
    return jnp.maximum(x, 0.0) + (jnp.exp(jnp.minimum(x, 0.0)) - 1.0)


def _b16(x):
    return x.astype(jnp.bfloat16)


def _mm(a, b):
    return jnp.dot(a, b, preferred_element_type=jnp.float32)


def _dotT(Wt, xb):
    return jax.lax.dot_general(Wt, xb, (((1,), (1,)), ((), ())),
                               preferred_element_type=jnp.float32)


def _fused_kernel(Aa_ref, Aw_ref, Ae_ref, Al_ref,
                  xT_ref, xTw_ref, xTe_ref, xD_ref, xDu_ref, xL_ref,
                  M1aT_ref, b1aT_ref, WV1sT_ref, b1sT_ref,
                  M2aT_ref, b2aT_ref, M2sT_ref, b2sT_ref,
                  L1T_ref, bl1T_ref, L2T_ref, bl2T_ref,
                  G8_ref, P8_ref, bfT_ref, Sel_ref,
                  out_ref):
    sel = Sel_ref[...]

    def prep(Aref):
        Ab = _b16(Aref[...]).reshape(_GM, _M)
        deg = _mm(sel, Ab)
        safe = jnp.where(deg > 0, deg, 1.0)
        d = jnp.where(deg > 0, jax.lax.rsqrt(safe), 0.0)
        return Ab, d

    Aa, da = prep(Aa_ref)
    Aw, dw = prep(Aw_ref)
    Ae, de = prep(Ae_ref)
    Al, dl = prep(Al_ref)

    def agg(Ab, d, hT):
        parts = []
        for i in range(_GP):
            di = d[i:i + 1]
            p = _mm(_b16(hT[:, i * _M:(i + 1) * _M] * di),
                    Ab[i * _M:(i + 1) * _M])
            parts.append(p * di)
        return jnp.concatenate(parts, axis=1)

    def rs(ref):
        return _b16(ref[...]).reshape(_GM, _F)

    xa = jnp.concatenate([rs(xT_ref), rs(xD_ref), rs(xDu_ref)], axis=1)
    xLb = rs(xL_ref)
    H1a = _dotT(M1aT_ref[...], xa)
    h1a = _elu(_gelu(agg(Aa, da, H1a[0:64]) + H1a[64:128] + b1aT_ref[...]))
    Hw = _dotT(WV1sT_ref[0], rs(xTw_ref))
    He = _dotT(WV1sT_ref[1], rs(xTe_ref))
    Hl = _dotT(WV1sT_ref[2], xLb)
    h1w = _elu(_gelu(agg(Aw, dw, Hw[0:32]) + Hw[32:64] + b1sT_ref[0]))
    h1e = _elu(_gelu(agg(Ae, de, He[0:32]) + He[32:64] + b1sT_ref[1]))
    h1l = _elu(_gelu(agg(Al, dl, Hl[0:32]) + Hl[32:64] + b1sT_ref[2]))

    H2a = _mm(M2aT_ref[...], _b16(h1a))
    s_a = jax.nn.relu(agg(Aa, da, H2a[0:24]) + H2a[24:48] + b2aT_ref[...])
    H2w = _mm(M2sT_ref[0], _b16(h1w))
    H2e = _mm(M2sT_ref[1], _b16(h1e))
    H2l = _mm(M2sT_ref[2], _b16(h1l))
    s_w = jax.nn.relu(agg(Aw, dw, H2w[0:8]) + H2w[8:16] + b2sT_ref[0])
    s_e = jax.nn.relu(agg(Ae, de, H2e[0:8]) + H2e[8:16] + b2sT_ref[1])
    s_l = jax.nn.relu(agg(Al, dl, H2l[0:8]) + H2l[8:16] + b2sT_ref[2])

    tT = jax.nn.relu(_dotT(L1T_ref[...], xLb) + bl1T_ref[...])
    s_lt = _mm(L2T_ref[...], _b16(tT)) + bl2T_ref[...]

    S = jnp.concatenate([s_a, s_w, s_e, s_l, s_lt], axis=0)
    E = jnp.exp(S - jnp.max(S, axis=0, keepdims=True))
    En = E / _mm(G8_ref[...], _b16(E))
    out_ref[...] = _mm(P8_ref[...], _b16(En)) + bfT_ref[...]


def _pad_set(shape, *placements):
    z = jnp.zeros(shape, jnp.float32)
    for (r, c), w in placements:
        z = jax.lax.dynamic_update_slice(z, w, (r, c))
    return z


def kernel(A_input, A_week_input, A_weekend_input, Location_location_input,
           Temporal_input, Temporal_week_input, Temporal_weekend_input,
           Distance_input, Duration_input, Location_time_input,
           W1_temporal, V1_temporal, b1_temporal, W2_temporal, V2_temporal, b2_temporal,
           W1_week, V1_week, b1_week, W2_week, V2_week, b2_week,
           W1_weekend, V1_weekend, b1_weekend, W2_weekend, V2_weekend, b2_weekend,
           W1_distance, V1_distance, b1_distance, W2_distance, V2_distance, b2_distance,
           W1_duration, V1_duration, b1_duration, W2_duration, V2_duration, b2_duration,
           W1_loctime, V1_loctime, b1_loctime, W2_loctime, V2_loctime, b2_loctime,
           L1, bl1, L2, bl2, Ld, bd, Lo, bo):
    f32 = jnp.float32
    M1aT = _pad_set((_M, 3 * _F),
                    ((0, 0), W1_temporal.T), ((20, 48), W1_distance.T),
                    ((40, 96), W1_duration.T),
                    ((64, 0), V1_temporal.T), ((84, 48), V1_distance.T),
                    ((104, 96), V1_duration.T))
    b1aT = _pad_set((64, 1), ((0, 0), jnp.concatenate(
        [b1_temporal, b1_distance, b1_duration])[:, None]))
    WV1sT = jnp.stack([
        _pad_set((64, _F), ((0, 0), W.T), ((32, 0), V.T))
        for W, V in [(W1_week, V1_week), (W1_weekend, V1_weekend),
                     (W1_loctime, V1_loctime)]])
    b1sT = jnp.stack([_pad_set((32, 1), ((0, 0), b[:, None]))
                      for b in [b1_week, b1_weekend, b1_loctime]])

    M2aT = _pad_set((48, 64),
                    ((0, 0), W2_temporal.T), ((8, 20), W2_distance.T),
                    ((16, 40), W2_duration.T),
                    ((24, 0), V2_temporal.T), ((32, 20), V2_distance.T),
                    ((40, 40), V2_duration.T))
    b2aT = _pad_set((24, 1), ((0, 0), b2_temporal[:, None]),
                    ((8, 0), b2_distance[:, None]),
                    ((16, 0), b2_duration[:, None]))
    M2sT = jnp.stack([
        _pad_set((16, 32), ((0, 0), W.T), ((8, 0), V.T))
        for W, V in [(W2_week, V2_week), (W2_weekend, V2_weekend),
                     (W2_loctime, V2_loctime)]])
    b2sT = jnp.stack([_pad_set((8, 1), ((0, 0), b[:, None]))
                      for b in [b2_week, b2_weekend, b2_loctime]])

    L1T = L1.T
    bl1T = bl1[:, None]
    L2T = _pad_set((8, 40), ((0, 0), L2.T))
    bl2T = _pad_set((8, 1), ((0, 0), bl2[:, None]))

    LoB = _pad_set((8, 8), ((0, 0), Lo.T))
    LdB = _pad_set((8, 8), ((0, 0), 2.0 * Ld.T))
    P8 = jnp.concatenate([LoB, LoB, LoB, LoB, LoB, LdB, LdB], axis=1)
    bfT = _pad_set((8, 1), ((0, 0), (bd + bo)[:, None]))

    grid = (_B // _GP,)
    badj = pl.BlockSpec((_GP, _M, _M), lambda b: (b, 0, 0))
    bx = pl.BlockSpec((_GP, _M, _F), lambda b: (b, 0, 0))

    def bcast(shape):
        nd = len(shape)
        return pl.BlockSpec(shape, lambda b: (0,) * nd)

    padded = pl.pallas_call(
        _fused_kernel,
        grid=grid,
        in_specs=[badj, badj, badj, badj,
                  bx, bx, bx, bx, bx, bx,
                  bcast((_M, 3 * _F)), bcast((64, 1)),
                  bcast((3, 64, _F)), bcast((3, 32, 1)),
                  bcast((48, 64)), bcast((24, 1)),
                  bcast((3, 16, 32)), bcast((3, 8, 1)),
                  bcast((40, _F)), bcast((40, 1)),
                  bcast((8, 40)), bcast((8, 1)),
                  bcast((56, 56)), bcast((8, 56)), bcast((8, 1)),
                  bcast((_GP, _GM))],
        out_specs=pl.BlockSpec((8, _GM), lambda b: (0, b)),
        out_shape=jax.ShapeDtypeStruct((8, _N), f32),
        compiler_params=pltpu.CompilerParams(
            dimension_semantics=("parallel",)),
    )(A_input, A_week_input, A_weekend_input, Location_location_input,
      Temporal_input, Temporal_week_input, Temporal_weekend_input,
      Distance_input, Duration_input, Location_time_input,
      _b16(M1aT), b1aT, _b16(WV1sT), b1sT,
      _b16(M2aT), b2aT, _b16(M2sT), b2sT,
      _b16(L1T), bl1T, _b16(L2T), bl2T,
      jnp.asarray(_G_BLOCK, dtype=jnp.bfloat16), _b16(P8), bfT,
      jnp.asarray(_SEL, dtype=jnp.bfloat16))
    return padded[:_C].T
